# baseline (speedup 1.0000x reference)
import functools

import jax
import jax.numpy as jnp
from jax import lax
from jax.experimental import pallas as pl
from jax.experimental.pallas import tpu as pltpu
from jax.experimental.pallas import tpu_sc as plsc

D = 128
L = 16
NC, NS = 1, 16
NW = NC * NS
C = 128
U = 8


P = L + 1


def kernel(head, relation, tail, entity_embedding, relation_embedding):
    B = head.shape[0]
    b_per_w = B // NW
    n_chunks = b_per_w // C

    mesh = plsc.VectorSubcoreMesh(
        core_axis_name="c", subcore_axis_name="s",
        num_cores=NC, num_subcores=NS)

    idx_t = pltpu.VMEM((C,), jnp.int32)
    rows_t = pltpu.VMEM((C, D), jnp.float32)

    @functools.partial(
        pl.kernel,
        out_type=jax.ShapeDtypeStruct((B,), jnp.float32),
        mesh=mesh,
        compiler_params=pltpu.CompilerParams(needs_layout_passes=False),
        scratch_types=[
            idx_t, idx_t, idx_t,
            idx_t, idx_t, idx_t,
            rows_t, rows_t, rows_t,
            rows_t, rows_t, rows_t,
            pltpu.VMEM((b_per_w,), jnp.float32),
            pltpu.VMEM((L * P,), jnp.float32),
            pltpu.SemaphoreType.DMA,
            pltpu.SemaphoreType.DMA,
        ],
    )
    def sc_kernel(head_r, rel_r, tail_r, ent_r, relemb_r, out_r,
                  hi0, ri0, ti0, hi1, ri1, ti1,
                  hr0, rr0, tr0, hr1, rr1, tr1,
                  score_v, part_v, sem0, sem1):
        wid = lax.axis_index("s") * NC + lax.axis_index("c")
        base_w = wid * b_per_w

        idx_refs = [(hi0, ri0, ti0), (hi1, ri1, ti1)]
        row_refs = [(hr0, rr0, tr0), (hr1, rr1, tr1)]
        sems = [sem0, sem1]

        def start(g, slot):
            hi, ri, ti = idx_refs[slot]
            hr, rr, tr = row_refs[slot]
            base = base_w + g * C
            pltpu.sync_copy(head_r.at[pl.ds(base, C)], hi)
            pltpu.sync_copy(rel_r.at[pl.ds(base, C)], ri)
            pltpu.sync_copy(tail_r.at[pl.ds(base, C)], ti)
            return [
                pltpu.async_copy(ent_r.at[hi], hr, sems[slot]),
                pltpu.async_copy(relemb_r.at[ri], rr, sems[slot]),
                pltpu.async_copy(ent_r.at[ti], tr, sems[slot]),
            ]

        colbase = lax.iota(jnp.int32, L) * P

        def compute(g, slot):
            hr, rr, tr = row_refs[slot]

            def group(j, carry):
                for e in range(L):
                    row = j * L + e
                    acc = jnp.zeros((L,), jnp.float32)
                    for c in range(D // L):
                        sl = pl.ds(c * L, L)
                        acc = acc + hr[row, sl] * rr[row, sl] * tr[row, sl]
                    part_v[pl.ds(e * P, L)] = acc
                tot = jnp.zeros((L,), jnp.float32)
                for i in range(L):
                    tot = tot + plsc.load_gather(part_v, [colbase + i])
                score_v[pl.ds(g * C + j * L, L)] = tot
                return carry

            lax.fori_loop(0, C // L, group, 0)

        handles = [None, None]
        handles[0] = start(0, 0)
        handles[1] = start(1, 1)
        for g in range(n_chunks):
            slot = g % 2
            for h in handles[slot]:
                h.wait()
            compute(g, slot)
            if g + 2 < n_chunks:
                handles[slot] = start(g + 2, slot)

        pltpu.sync_copy(score_v, out_r.at[pl.ds(base_w, b_per_w)])

    return sc_kernel(head, relation, tail, entity_embedding, relation_embedding)


# speedup vs baseline: 1.4674x; 1.4674x over previous
import functools

import jax
import jax.numpy as jnp
from jax import lax
from jax.experimental import pallas as pl
from jax.experimental.pallas import tpu as pltpu
from jax.experimental.pallas import tpu_sc as plsc

D = 128
L = 16
NC, NS = 2, 16
NW = NC * NS
C = 128
P = L + 1


def kernel(head, relation, tail, entity_embedding, relation_embedding):
    B = head.shape[0]
    R = relation_embedding.shape[0]
    b_per_w = B // NW
    n_chunks = b_per_w // C

    mesh = plsc.VectorSubcoreMesh(
        core_axis_name="c", subcore_axis_name="s",
        num_cores=NC, num_subcores=NS)

    idx_t = pltpu.VMEM((b_per_w,), jnp.int32)
    rows_t = pltpu.VMEM((C, D), jnp.float32)

    @functools.partial(
        pl.kernel,
        out_type=jax.ShapeDtypeStruct((B,), jnp.float32),
        mesh=mesh,
        compiler_params=pltpu.CompilerParams(needs_layout_passes=False),
        scratch_types=[
            idx_t, idx_t, idx_t,
            rows_t, rows_t, rows_t,
            rows_t, rows_t, rows_t,
            pltpu.VMEM((b_per_w,), jnp.float32),
            pltpu.VMEM((L * P,), jnp.float32),
            pltpu.SemaphoreType.DMA,
            pltpu.SemaphoreType.DMA,
        ],
    )
    def sc_kernel(head_r, rel_r, tail_r, ent_r, relemb_r, out_r,
                  hi_v, ri_v, ti_v,
                  hr0, rr0, tr0, hr1, rr1, tr1,
                  score_v, part_v, sem0, sem1):
        sid = lax.axis_index("s")
        wid = sid * NC + lax.axis_index("c")
        base_w = wid * b_per_w

        pltpu.sync_copy(head_r.at[pl.ds(base_w, b_per_w)], hi_v)
        pltpu.sync_copy(rel_r.at[pl.ds(base_w, b_per_w)], ri_v)
        pltpu.sync_copy(tail_r.at[pl.ds(base_w, b_per_w)], ti_v)

        row_refs = [(hr0, rr0, tr0), (hr1, rr1, tr1)]
        sems = [sem0, sem1]

        def start(g, slot):
            hr, rr, tr = row_refs[slot]
            sl = pl.ds(g * C, C)
            return [
                pltpu.async_copy(ent_r.at[hi_v.at[sl]], hr, sems[slot]),
                pltpu.async_copy(relemb_r.at[ri_v.at[sl]], rr, sems[slot]),
                pltpu.async_copy(ent_r.at[ti_v.at[sl]], tr, sems[slot]),
            ]

        colbase = lax.iota(jnp.int32, L) * P

        def compute(g, slot):
            hr, rr, tr = row_refs[slot]

            def group(j, carry):
                for e in range(L):
                    row = j * L + e
                    acc = jnp.zeros((L,), jnp.float32)
                    for c in range(D // L):
                        sl = pl.ds(c * L, L)
                        acc = acc + hr[row, sl] * rr[row, sl] * tr[row, sl]
                    part_v[pl.ds(e * P, L)] = acc
                tot = jnp.zeros((L,), jnp.float32)
                for i in range(L):
                    tot = tot + plsc.load_gather(part_v, [colbase + i])
                score_v[pl.ds(g * C + j * L, L)] = tot
                return carry

            lax.fori_loop(0, C // L, group, 0)

        handles = [None, None]
        handles[0] = start(0, 0)
        handles[1] = start(1, 1)
        for g in range(n_chunks):
            slot = g % 2
            for h in handles[slot]:
                h.wait()
            compute(g, slot)
            if g + 2 < n_chunks:
                handles[slot] = start(g + 2, slot)

        pltpu.sync_copy(score_v, out_r.at[pl.ds(base_w, b_per_w)])

    return sc_kernel(head, relation, tail, entity_embedding, relation_embedding)
